# baseline (speedup 1.0000x reference)
import dataclasses

import jax
import jax.numpy as jnp
from jax import lax
from jax.experimental import pallas as pl
from jax.experimental.pallas import tpu as pltpu
from jax.experimental.pallas import tpu_sc as plsc

NC = 2
NS = 16
NW = NC * NS
L = 16
PADC = 16

_mesh = plsc.VectorSubcoreMesh(
    core_axis_name="c", subcore_axis_name="s", num_cores=NC, num_subcores=NS
)

_cp = pltpu.CompilerParams(
    needs_layout_passes=False, use_tc_tiling_on_sc=False
)


def _chunk(epw: int) -> int:
    for c in range(min(2048, epw), 0, -16):
        if epw % c == 0:
            return c
    raise ValueError(f"no 16-multiple chunk divides {epw}")


def _chunk_unroll(epw: int, want_c: int, un: int):
    for c in range(min(want_c, epw), 0, -16):
        if epw % c == 0 and (epw // c) % un == 0:
            return c, un, epw // c // un
    return _chunk(epw), 1, epw // _chunk(epw)


def _pad_rows(n: int) -> int:
    per = -(-n // NS)
    per = -(-per // 8) * 8
    return per * NS


def _pieces(total: int, c: int):
    out, off = [], 0
    while off < total:
        sz = min(c, total - off)
        out.append((off, sz))
        off += sz
    return out


def _worker():
    cid = lax.axis_index("c")
    sid = lax.axis_index("s")
    return cid, sid, sid * NC + cid


def _deg_call(vrow, zeros_rn, RN, E):
    EPW = E // NW
    C, UN, OUT = _chunk_unroll(EPW, 1024, 5)
    PS = RN // NS
    assert RN % (NS * 8) == 0

    def body(vrow_hbm, z_hbm, deg_out, deg_sh, ones_v, stage_v, *rest):
        idx_v = rest[:UN]
        sems = rest[UN:]
        cid, sid, wid = _worker()
        for off, sz in _pieces(PS, C):
            pltpu.sync_copy(z_hbm.at[pl.ds(sid * PS + off, sz)],
                            stage_v.at[pl.ds(0, sz)])
            pltpu.sync_copy(stage_v.at[pl.ds(0, sz)],
                            deg_sh.at[pl.ds(sid * PS + off, sz)])

        @pl.loop(0, C, step=L)
        def _(i):
            ones_v[pl.ds(i, L)] = jnp.full((L,), 1.0, jnp.float32)

        plsc.subcore_barrier()

        @pl.loop(0, OUT)
        def _(g):
            base0 = wid * EPW + g * (UN * C)
            loads = [pltpu.async_copy(
                vrow_hbm.at[pl.ds(base0 + b * C, C)], idx_v[b], sems[b])
                for b in range(UN)]
            scats = []
            for b in range(UN):
                loads[b].wait()
                scats.append(pltpu.async_copy(
                    ones_v, deg_sh.at[idx_v[b]], sems[b], add=True))
            for d in scats:
                d.wait()

        plsc.subcore_barrier()
        for off, sz in _pieces(PS, C):
            pltpu.sync_copy(deg_sh.at[pl.ds(sid * PS + off, sz)],
                            stage_v.at[pl.ds(0, sz)])
            pltpu.sync_copy(stage_v.at[pl.ds(0, sz)],
                            deg_out.at[pl.ds(cid * RN + sid * PS + off, sz)])

    call = pl.kernel(
        body,
        out_type=jax.ShapeDtypeStruct((NC * RN,), jnp.float32),
        mesh=_mesh,
        compiler_params=_cp,
        scratch_types=(
            [pltpu.VMEM_SHARED((RN,), jnp.float32),
             pltpu.VMEM((C,), jnp.float32),
             pltpu.VMEM((C,), jnp.float32)]
            + [pltpu.VMEM((C,), jnp.int32) for _ in range(UN)]
            + [pltpu.SemaphoreType.DMA for _ in range(UN)]
        ),
    )
    return call(vrow, zeros_rn)


def _h_call(vrow, hidx, srcw, deg0, deg1, w1, zeros_h, NP, E, EMB):
    EPW = E // NW
    C, UN, OUT = _chunk_unroll(EPW, 400, 5)
    PS = NP // NS

    def body(vrow_hbm, hidx_hbm, src_hbm, d0_hbm, d1_hbm, w1_hbm, zh_hbm,
             h_out, vals_out,
             h_sh, *rest):
        vr = rest[0:UN]
        hi = rest[UN:2 * UN]
        si = rest[2 * UN:3 * UN]
        d0 = rest[3 * UN:4 * UN]
        d1 = rest[4 * UN:5 * UN]
        va = rest[5 * UN:6 * UN]
        rows = rest[6 * UN:7 * UN]
        sems = rest[7 * UN:]
        cid, sid, wid = _worker()
        for off, sz in _pieces(PS, C):
            pltpu.sync_copy(zh_hbm.at[pl.ds(sid * PS + off, sz), :],
                            rows[0].at[pl.ds(0, sz), :])
            pltpu.sync_copy(rows[0].at[pl.ds(0, sz), :],
                            h_sh.at[pl.ds(sid * PS + off, sz), :])
        plsc.subcore_barrier()

        @pl.loop(0, OUT)
        def _(g):
            base0 = wid * EPW + g * (UN * C)
            loads = []
            for b in range(UN):
                s = pl.ds(base0 + b * C, C)
                loads.append([
                    pltpu.async_copy(vrow_hbm.at[s], vr[b], sems[b]),
                    pltpu.async_copy(hidx_hbm.at[s], hi[b], sems[b]),
                    pltpu.async_copy(src_hbm.at[s], si[b], sems[b]),
                ])
            gaths = []
            for b in range(UN):
                for d in loads[b]:
                    d.wait()
                gaths.append([
                    pltpu.async_copy(d0_hbm.at[vr[b]], d0[b], sems[b]),
                    pltpu.async_copy(d1_hbm.at[vr[b]], d1[b], sems[b]),
                    pltpu.async_copy(w1_hbm.at[hi[b]], rows[b], sems[b]),
                ])
            fins = []
            for b in range(UN):
                for d in gaths[b]:
                    d.wait()

                @pl.loop(0, C, step=L)
                def _(i):
                    s = pl.ds(i, L)
                    va[b][s] = 1.0 / (d0[b][s] + d1[b][s])

                fins.append(pltpu.async_copy(
                    va[b], vals_out.at[pl.ds(base0 + b * C, C)], sems[b]))

                @pl.loop(0, C)
                def _(i):
                    vv = plsc.load_gather(
                        va[b], [jnp.full((L,), i, jnp.int32)])
                    rows[b][i, :] = rows[b][i, :] * vv

                fins.append(pltpu.async_copy(
                    rows[b], h_sh.at[si[b]], sems[b], add=True))
            for d in fins:
                d.wait()

        plsc.subcore_barrier()
        for off, sz in _pieces(PS, C):
            pltpu.sync_copy(h_sh.at[pl.ds(sid * PS + off, sz), :],
                            rows[0].at[pl.ds(0, sz), :])
            pltpu.sync_copy(rows[0].at[pl.ds(0, sz), :],
                            h_out.at[pl.ds(cid * NP + sid * PS + off, sz), :])

    call = pl.kernel(
        body,
        out_type=(
            jax.ShapeDtypeStruct((NC * NP, EMB), jnp.float32),
            jax.ShapeDtypeStruct((E,), jnp.float32),
        ),
        mesh=_mesh,
        compiler_params=_cp,
        scratch_types=(
            [pltpu.VMEM_SHARED((NP, EMB), jnp.float32)]
            + [pltpu.VMEM((C,), jnp.int32) for _ in range(3 * UN)]
            + [pltpu.VMEM((C,), jnp.float32) for _ in range(3 * UN)]
            + [pltpu.VMEM((C, EMB), jnp.float32) for _ in range(UN)]
            + [pltpu.SemaphoreType.DMA for _ in range(UN)]
        ),
    )
    return call(vrow, hidx, srcw, deg0, deg1, w1, zeros_h)


def _g_call(h_parts, b1, W, N, EMB, RW):
    BN = 2000
    assert N % BN == 0

    def body(h_ref, b1_ref, w_ref, g_ref):
        h = h_ref[0] + h_ref[1] + b1_ref[...]
        h = jnp.maximum(h, 0.0)
        g_ref[...] = lax.dot_general(
            h, w_ref[...], (((1,), (0,)), ((), ())),
            precision=lax.Precision.HIGHEST,
            preferred_element_type=jnp.float32,
        )

    return pl.pallas_call(
        body,
        grid=(N // BN,),
        in_specs=[
            pl.BlockSpec((NC, BN, EMB), lambda i: (0, i, 0)),
            pl.BlockSpec((1, EMB), lambda i: (0, 0)),
            pl.BlockSpec((EMB, RW), lambda i: (0, 0)),
        ],
        out_specs=pl.BlockSpec((BN, RW), lambda i: (i, 0)),
        out_shape=jax.ShapeDtypeStruct((N, RW), jnp.float32),
    )(h_parts, b1, W)


def _out_call(gidx, srcw, vals, g2, zeros_o, NP, E):
    EPW = E // NW
    C, UN, OUT = _chunk_unroll(EPW, 400, 5)
    PS = NP // NS

    def body(gidx_hbm, src_hbm, vals_hbm, g_hbm, zo_hbm,
             o_out,
             o_sh, *rest):
        gi = rest[0:UN]
        si = rest[UN:2 * UN]
        va = rest[2 * UN:3 * UN]
        rows = rest[3 * UN:4 * UN]
        sems = rest[4 * UN:]
        cid, sid, wid = _worker()
        for off, sz in _pieces(PS, C):
            pltpu.sync_copy(zo_hbm.at[pl.ds(sid * PS + off, sz), :],
                            rows[0].at[pl.ds(0, sz), :])
            pltpu.sync_copy(rows[0].at[pl.ds(0, sz), :],
                            o_sh.at[pl.ds(sid * PS + off, sz), :])
        plsc.subcore_barrier()

        @pl.loop(0, OUT)
        def _(g):
            base0 = wid * EPW + g * (UN * C)
            loads = []
            for b in range(UN):
                s = pl.ds(base0 + b * C, C)
                loads.append([
                    pltpu.async_copy(gidx_hbm.at[s], gi[b], sems[b]),
                    pltpu.async_copy(src_hbm.at[s], si[b], sems[b]),
                    pltpu.async_copy(vals_hbm.at[s], va[b], sems[b]),
                ])
            gaths = []
            for b in range(UN):
                for d in loads[b]:
                    d.wait()
                gaths.append(
                    pltpu.async_copy(g_hbm.at[gi[b]], rows[b], sems[b]))
            fins = []
            for b in range(UN):
                gaths[b].wait()

                @pl.loop(0, C)
                def _(i):
                    vv = plsc.load_gather(
                        va[b], [jnp.full((L,), i, jnp.int32)])
                    rows[b][i, :] = rows[b][i, :] * vv

                fins.append(pltpu.async_copy(
                    rows[b], o_sh.at[si[b]], sems[b], add=True))
            for d in fins:
                d.wait()

        plsc.subcore_barrier()
        for off, sz in _pieces(PS, C):
            pltpu.sync_copy(o_sh.at[pl.ds(sid * PS + off, sz), :],
                            rows[0].at[pl.ds(0, sz), :])
            pltpu.sync_copy(rows[0].at[pl.ds(0, sz), :],
                            o_out.at[pl.ds(cid * NP + sid * PS + off, sz), :])

    call = pl.kernel(
        body,
        out_type=jax.ShapeDtypeStruct((NC * NP, PADC), jnp.float32),
        mesh=_mesh,
        compiler_params=_cp,
        scratch_types=(
            [pltpu.VMEM_SHARED((NP, PADC), jnp.float32)]
            + [pltpu.VMEM((C,), jnp.int32) for _ in range(2 * UN)]
            + [pltpu.VMEM((C,), jnp.float32) for _ in range(UN)]
            + [pltpu.VMEM((C, PADC), jnp.float32) for _ in range(UN)]
            + [pltpu.SemaphoreType.DMA for _ in range(UN)]
        ),
    )
    return call(gidx, srcw, vals, g2, zeros_o)


def kernel(src, dst, rel, weights1, bias1, weights2, bias2):
    R, N, EMB = weights1.shape
    NUMCLS = weights2.shape[-1]
    E = src.shape[0]
    RN = R * N
    NP = _pad_rows(N)

    src = src.astype(jnp.int32)
    dst = dst.astype(jnp.int32)
    rel = rel.astype(jnp.int32)
    vrow = rel * N + src
    hidx = rel * N + dst
    gidx = dst * R + rel

    w1 = weights1.reshape(RN, EMB)
    W = jnp.zeros((R, EMB, PADC), jnp.float32).at[:, :, :NUMCLS].set(weights2)
    W = W.transpose(1, 0, 2).reshape(EMB, R * PADC)

    zeros_rn = jnp.zeros((RN,), jnp.float32)
    zeros_h = jnp.zeros((NP, EMB), jnp.float32)
    zeros_o = jnp.zeros((NP, PADC), jnp.float32)

    deg_parts = _deg_call(vrow, zeros_rn, RN, E)
    h_parts, vals = _h_call(vrow, hidx, src, deg_parts[:RN], deg_parts[RN:],
                            w1, zeros_h, NP, E, EMB)
    hp = h_parts.reshape(NC, NP, EMB)[:, :N, :]
    g = _g_call(hp, bias1.reshape(1, EMB), W, N, EMB, R * PADC)
    g2 = g.reshape(N * R, PADC)
    o_parts = _out_call(gidx, src, vals, g2, zeros_o, NP, E)
    op = o_parts.reshape(NC, NP, PADC)
    out = op[0, :N, :NUMCLS] + op[1, :N, :NUMCLS] + bias2
    return out


# speedup vs baseline: 1.0101x; 1.0101x over previous
import dataclasses

import jax
import jax.numpy as jnp
from jax import lax
from jax.experimental import pallas as pl
from jax.experimental.pallas import tpu as pltpu
from jax.experimental.pallas import tpu_sc as plsc

NC = 2
NS = 16
NW = NC * NS
L = 16
PADC = 16

_mesh = plsc.VectorSubcoreMesh(
    core_axis_name="c", subcore_axis_name="s", num_cores=NC, num_subcores=NS
)

_cp = pltpu.CompilerParams(
    needs_layout_passes=False, use_tc_tiling_on_sc=False
)


def _chunk(epw: int) -> int:
    for c in range(min(2048, epw), 0, -16):
        if epw % c == 0:
            return c
    raise ValueError(f"no 16-multiple chunk divides {epw}")


def _chunk_unroll(epw: int, want_c: int, un: int):
    for c in range(min(want_c, epw), 0, -16):
        if epw % c == 0 and (epw // c) % un == 0:
            return c, un, epw // c // un
    return _chunk(epw), 1, epw // _chunk(epw)


def _pad_rows(n: int) -> int:
    per = -(-n // NS)
    per = -(-per // 8) * 8
    return per * NS


def _pieces(total: int, c: int):
    out, off = [], 0
    while off < total:
        sz = min(c, total - off)
        out.append((off, sz))
        off += sz
    return out


def _worker():
    cid = lax.axis_index("c")
    sid = lax.axis_index("s")
    return cid, sid, sid * NC + cid


def _deg_call(vrow, zeros_rn, RN, E):
    EPW = E // NW
    C = _chunk(EPW)
    NCH = EPW // C
    PS = RN // NS
    assert RN % (NS * 8) == 0

    def body(vrow_hbm, z_hbm, deg_out, deg_sh, ones_v, stage_v, idx_v):
        cid, sid, wid = _worker()
        for off, sz in _pieces(PS, C):
            pltpu.sync_copy(z_hbm.at[pl.ds(sid * PS + off, sz)],
                            stage_v.at[pl.ds(0, sz)])
            pltpu.sync_copy(stage_v.at[pl.ds(0, sz)],
                            deg_sh.at[pl.ds(sid * PS + off, sz)])

        @pl.loop(0, C, step=L)
        def _(i):
            ones_v[pl.ds(i, L)] = jnp.full((L,), 1.0, jnp.float32)

        plsc.subcore_barrier()

        @pl.loop(0, NCH)
        def _(ch):
            base = wid * EPW + ch * C
            pltpu.sync_copy(vrow_hbm.at[pl.ds(base, C)], idx_v)
            pltpu.sync_copy(ones_v, deg_sh.at[idx_v], add=True)

        plsc.subcore_barrier()
        for off, sz in _pieces(PS, C):
            pltpu.sync_copy(deg_sh.at[pl.ds(sid * PS + off, sz)],
                            stage_v.at[pl.ds(0, sz)])
            pltpu.sync_copy(stage_v.at[pl.ds(0, sz)],
                            deg_out.at[pl.ds(cid * RN + sid * PS + off, sz)])

    call = pl.kernel(
        body,
        out_type=jax.ShapeDtypeStruct((NC * RN,), jnp.float32),
        mesh=_mesh,
        compiler_params=_cp,
        scratch_types=[
            pltpu.VMEM_SHARED((RN,), jnp.float32),
            pltpu.VMEM((C,), jnp.float32),
            pltpu.VMEM((C,), jnp.float32),
            pltpu.VMEM((C,), jnp.int32),
        ],
    )
    return call(vrow, zeros_rn)


def _h_call(vrow, hidx, srcw, deg0, deg1, w1, zeros_h, NP, E, EMB):
    EPW = E // NW
    C, UN, OUT = _chunk_unroll(EPW, 400, 5)
    PS = NP // NS

    def body(vrow_hbm, hidx_hbm, src_hbm, d0_hbm, d1_hbm, w1_hbm, zh_hbm,
             h_out, vals_out,
             h_sh, *rest):
        vr = rest[0:UN]
        hi = rest[UN:2 * UN]
        si = rest[2 * UN:3 * UN]
        d0 = rest[3 * UN:4 * UN]
        d1 = rest[4 * UN:5 * UN]
        va = rest[5 * UN:6 * UN]
        rows = rest[6 * UN:7 * UN]
        sems = rest[7 * UN:]
        cid, sid, wid = _worker()
        for off, sz in _pieces(PS, C):
            pltpu.sync_copy(zh_hbm.at[pl.ds(sid * PS + off, sz), :],
                            rows[0].at[pl.ds(0, sz), :])
            pltpu.sync_copy(rows[0].at[pl.ds(0, sz), :],
                            h_sh.at[pl.ds(sid * PS + off, sz), :])
        plsc.subcore_barrier()

        @pl.loop(0, OUT)
        def _(g):
            base0 = wid * EPW + g * (UN * C)
            loads = []
            for b in range(UN):
                s = pl.ds(base0 + b * C, C)
                loads.append([
                    pltpu.async_copy(vrow_hbm.at[s], vr[b], sems[b]),
                    pltpu.async_copy(hidx_hbm.at[s], hi[b], sems[b]),
                    pltpu.async_copy(src_hbm.at[s], si[b], sems[b]),
                ])
            gaths = []
            for b in range(UN):
                for d in loads[b]:
                    d.wait()
                gaths.append([
                    pltpu.async_copy(d0_hbm.at[vr[b]], d0[b], sems[b]),
                    pltpu.async_copy(d1_hbm.at[vr[b]], d1[b], sems[b]),
                    pltpu.async_copy(w1_hbm.at[hi[b]], rows[b], sems[b]),
                ])
            fins = []
            for b in range(UN):
                for d in gaths[b]:
                    d.wait()

                @pl.loop(0, C, step=L)
                def _(i):
                    s = pl.ds(i, L)
                    va[b][s] = 1.0 / (d0[b][s] + d1[b][s])

                fins.append(pltpu.async_copy(
                    va[b], vals_out.at[pl.ds(base0 + b * C, C)], sems[b]))

                @pl.loop(0, C)
                def _(i):
                    vv = plsc.load_gather(
                        va[b], [jnp.full((L,), i, jnp.int32)])
                    rows[b][i, :] = rows[b][i, :] * vv

                fins.append(pltpu.async_copy(
                    rows[b], h_sh.at[si[b]], sems[b], add=True))
            for d in fins:
                d.wait()

        plsc.subcore_barrier()
        for off, sz in _pieces(PS, C):
            pltpu.sync_copy(h_sh.at[pl.ds(sid * PS + off, sz), :],
                            rows[0].at[pl.ds(0, sz), :])
            pltpu.sync_copy(rows[0].at[pl.ds(0, sz), :],
                            h_out.at[pl.ds(cid * NP + sid * PS + off, sz), :])

    call = pl.kernel(
        body,
        out_type=(
            jax.ShapeDtypeStruct((NC * NP, EMB), jnp.float32),
            jax.ShapeDtypeStruct((E,), jnp.float32),
        ),
        mesh=_mesh,
        compiler_params=_cp,
        scratch_types=(
            [pltpu.VMEM_SHARED((NP, EMB), jnp.float32)]
            + [pltpu.VMEM((C,), jnp.int32) for _ in range(3 * UN)]
            + [pltpu.VMEM((C,), jnp.float32) for _ in range(3 * UN)]
            + [pltpu.VMEM((C, EMB), jnp.float32) for _ in range(UN)]
            + [pltpu.SemaphoreType.DMA for _ in range(UN)]
        ),
    )
    return call(vrow, hidx, srcw, deg0, deg1, w1, zeros_h)


def _g_call(h_parts, b1, W, N, EMB, RW):
    BN = 2000
    assert N % BN == 0 and RW % 128 == 0
    NQ = RW // 128

    def body(h_ref, b1_ref, w_ref, g_ref):
        h = h_ref[0] + h_ref[1] + b1_ref[...]
        h = jnp.maximum(h, 0.0)
        g_ref[...] = lax.dot_general(
            h, w_ref[...], (((1,), (0,)), ((), ())),
            precision=lax.Precision.HIGHEST,
            preferred_element_type=jnp.float32,
        )

    return pl.pallas_call(
        body,
        grid=(NQ, N // BN),
        in_specs=[
            pl.BlockSpec((NC, BN, EMB), lambda q, i: (0, i, 0)),
            pl.BlockSpec((1, EMB), lambda q, i: (0, 0)),
            pl.BlockSpec((EMB, 128), lambda q, i: (0, q)),
        ],
        out_specs=pl.BlockSpec((BN, 128), lambda q, i: (q * (N // BN) + i, 0)),
        out_shape=jax.ShapeDtypeStruct((NQ * N, 128), jnp.float32),
    )(h_parts, b1, W)


def _out_call(gidx, srcw, vals, g2, zeros_o, NP, E):
    EPW = E // NW
    C, UN, OUT = _chunk_unroll(EPW, 400, 5)
    PS = NP // NS

    def body(gidx_hbm, src_hbm, vals_hbm, g_hbm, zo_hbm,
             o_out,
             o_sh, *rest):
        gi = rest[0:UN]
        si = rest[UN:2 * UN]
        va = rest[2 * UN:3 * UN]
        rows = rest[3 * UN:4 * UN]
        sems = rest[4 * UN:]
        cid, sid, wid = _worker()
        for off, sz in _pieces(PS, C):
            pltpu.sync_copy(zo_hbm.at[pl.ds(sid * PS + off, sz), :],
                            rows[0].at[pl.ds(0, sz), :])
            pltpu.sync_copy(rows[0].at[pl.ds(0, sz), :],
                            o_sh.at[pl.ds(sid * PS + off, sz), :])
        plsc.subcore_barrier()

        @pl.loop(0, OUT)
        def _(g):
            base0 = wid * EPW + g * (UN * C)
            loads = []
            for b in range(UN):
                s = pl.ds(base0 + b * C, C)
                loads.append([
                    pltpu.async_copy(gidx_hbm.at[s], gi[b], sems[b]),
                    pltpu.async_copy(src_hbm.at[s], si[b], sems[b]),
                    pltpu.async_copy(vals_hbm.at[s], va[b], sems[b]),
                ])
            gaths = []
            for b in range(UN):
                for d in loads[b]:
                    d.wait()
                gaths.append(
                    pltpu.async_copy(g_hbm.at[gi[b]], rows[b], sems[b]))
            fins = []
            for b in range(UN):
                gaths[b].wait()

                @pl.loop(0, C)
                def _(i):
                    vv = plsc.load_gather(
                        va[b], [jnp.full((L,), i, jnp.int32)])
                    rows[b][i, :] = rows[b][i, :] * vv

                fins.append(pltpu.async_copy(
                    rows[b], o_sh.at[si[b]], sems[b], add=True))
            for d in fins:
                d.wait()

        plsc.subcore_barrier()
        for off, sz in _pieces(PS, C):
            pltpu.sync_copy(o_sh.at[pl.ds(sid * PS + off, sz), :],
                            rows[0].at[pl.ds(0, sz), :])
            pltpu.sync_copy(rows[0].at[pl.ds(0, sz), :],
                            o_out.at[pl.ds(cid * NP + sid * PS + off, sz), :])

    call = pl.kernel(
        body,
        out_type=jax.ShapeDtypeStruct((NC * NP, PADC), jnp.float32),
        mesh=_mesh,
        compiler_params=_cp,
        scratch_types=(
            [pltpu.VMEM_SHARED((NP, PADC), jnp.float32)]
            + [pltpu.VMEM((C,), jnp.int32) for _ in range(2 * UN)]
            + [pltpu.VMEM((C,), jnp.float32) for _ in range(UN)]
            + [pltpu.VMEM((C, PADC), jnp.float32) for _ in range(UN)]
            + [pltpu.SemaphoreType.DMA for _ in range(UN)]
        ),
    )
    return call(gidx, srcw, vals, g2, zeros_o)


def kernel(src, dst, rel, weights1, bias1, weights2, bias2):
    R, N, EMB = weights1.shape
    NUMCLS = weights2.shape[-1]
    E = src.shape[0]
    RN = R * N
    NP = _pad_rows(N)

    src = src.astype(jnp.int32)
    dst = dst.astype(jnp.int32)
    rel = rel.astype(jnp.int32)
    vrow = rel * N + src
    hidx = rel * N + dst
    RPQ = 128 // PADC
    gidx = (rel // RPQ) * (N * RPQ) + dst * RPQ + rel % RPQ

    w1 = weights1.reshape(RN, EMB)
    W = jnp.zeros((R, EMB, PADC), jnp.float32).at[:, :, :NUMCLS].set(weights2)
    W = W.transpose(1, 0, 2).reshape(EMB, R * PADC)

    zeros_rn = jnp.zeros((RN,), jnp.float32)
    zeros_h = jnp.zeros((NP, EMB), jnp.float32)
    zeros_o = jnp.zeros((NP, PADC), jnp.float32)

    deg_parts = _deg_call(vrow, zeros_rn, RN, E)
    h_parts, vals = _h_call(vrow, hidx, src, deg_parts[:RN], deg_parts[RN:],
                            w1, zeros_h, NP, E, EMB)
    hp = h_parts.reshape(NC, NP, EMB)[:, :N, :]
    g = _g_call(hp, bias1.reshape(1, EMB), W, N, EMB, R * PADC)
    g2 = g.reshape(N * R, PADC)
    o_parts = _out_call(gidx, src, vals, g2, zeros_o, NP, E)
    op = o_parts.reshape(NC, NP, PADC)
    out = op[0, :N, :NUMCLS] + op[1, :N, :NUMCLS] + bias2
    return out


# speedup vs baseline: 1.1121x; 1.1010x over previous
import dataclasses

import jax
import jax.numpy as jnp
from jax import lax
from jax.experimental import pallas as pl
from jax.experimental.pallas import tpu as pltpu
from jax.experimental.pallas import tpu_sc as plsc

NC = 2
NS = 16
NW = NC * NS
L = 16
PADC = 16

_mesh = plsc.VectorSubcoreMesh(
    core_axis_name="c", subcore_axis_name="s", num_cores=NC, num_subcores=NS
)

_cp = pltpu.CompilerParams(
    needs_layout_passes=False, use_tc_tiling_on_sc=False
)


def _chunk(epw: int) -> int:
    for c in range(min(2048, epw), 0, -16):
        if epw % c == 0:
            return c
    raise ValueError(f"no 16-multiple chunk divides {epw}")


def _chunk_unroll(epw: int, want_c: int, un: int):
    for c in range(min(want_c, epw), 0, -16):
        if epw % c == 0 and (epw // c) % un == 0:
            return c, un, epw // c // un
    return _chunk(epw), 1, epw // _chunk(epw)


def _pad_rows(n: int) -> int:
    per = -(-n // NS)
    per = -(-per // 8) * 8
    return per * NS


def _pieces(total: int, c: int):
    out, off = [], 0
    while off < total:
        sz = min(c, total - off)
        out.append((off, sz))
        off += sz
    return out


def _worker():
    cid = lax.axis_index("c")
    sid = lax.axis_index("s")
    return cid, sid, sid * NC + cid


def _deg_call(vrow, zeros_rn, RN, E):
    EPW = E // NW
    C = _chunk(EPW)
    NCH = EPW // C
    PS = RN // NS
    assert RN % (NS * 8) == 0

    def body(vrow_hbm, z_hbm, deg_out, deg_sh, ones_v, stage_v, idx_v):
        cid, sid, wid = _worker()
        for off, sz in _pieces(PS, C):
            pltpu.sync_copy(z_hbm.at[pl.ds(sid * PS + off, sz)],
                            stage_v.at[pl.ds(0, sz)])
            pltpu.sync_copy(stage_v.at[pl.ds(0, sz)],
                            deg_sh.at[pl.ds(sid * PS + off, sz)])

        @pl.loop(0, C, step=L)
        def _(i):
            ones_v[pl.ds(i, L)] = jnp.full((L,), 1.0, jnp.float32)

        plsc.subcore_barrier()

        @pl.loop(0, NCH)
        def _(ch):
            base = wid * EPW + ch * C
            pltpu.sync_copy(vrow_hbm.at[pl.ds(base, C)], idx_v)
            pltpu.sync_copy(ones_v, deg_sh.at[idx_v], add=True)

        plsc.subcore_barrier()
        for off, sz in _pieces(PS, C):
            pltpu.sync_copy(deg_sh.at[pl.ds(sid * PS + off, sz)],
                            stage_v.at[pl.ds(0, sz)])
            pltpu.sync_copy(stage_v.at[pl.ds(0, sz)],
                            deg_out.at[pl.ds(cid * RN + sid * PS + off, sz)])

    call = pl.kernel(
        body,
        out_type=jax.ShapeDtypeStruct((NC * RN,), jnp.float32),
        mesh=_mesh,
        compiler_params=_cp,
        scratch_types=[
            pltpu.VMEM_SHARED((RN,), jnp.float32),
            pltpu.VMEM((C,), jnp.float32),
            pltpu.VMEM((C,), jnp.float32),
            pltpu.VMEM((C,), jnp.int32),
        ],
    )
    return call(vrow, zeros_rn)


def _h_call(vrow, hidx, srcw, deg0, deg1, w1, zeros_h, NP, E, EMB):
    EPW = E // NW
    C, UN, OUT = _chunk_unroll(EPW, 400, 5)
    PS = NP // NS

    def body(vrow_hbm, hidx_hbm, src_hbm, d0_hbm, d1_hbm, w1_hbm, zh_hbm,
             h_out, vals_out,
             h_sh, *rest):
        vr = rest[0:UN]
        hi = rest[UN:2 * UN]
        si = rest[2 * UN:3 * UN]
        d0 = rest[3 * UN:4 * UN]
        d1 = rest[4 * UN:5 * UN]
        va = rest[5 * UN:6 * UN]
        rows = rest[6 * UN:7 * UN]
        sems = rest[7 * UN:]
        cid, sid, wid = _worker()
        for off, sz in _pieces(PS, C):
            pltpu.sync_copy(zh_hbm.at[pl.ds(sid * PS + off, sz), :],
                            rows[0].at[pl.ds(0, sz), :])
            pltpu.sync_copy(rows[0].at[pl.ds(0, sz), :],
                            h_sh.at[pl.ds(sid * PS + off, sz), :])
        plsc.subcore_barrier()

        @pl.loop(0, OUT)
        def _(g):
            base0 = wid * EPW + g * (UN * C)
            loads = []
            for b in range(UN):
                s = pl.ds(base0 + b * C, C)
                loads.append([
                    pltpu.async_copy(vrow_hbm.at[s], vr[b], sems[b]),
                    pltpu.async_copy(hidx_hbm.at[s], hi[b], sems[b]),
                    pltpu.async_copy(src_hbm.at[s], si[b], sems[b]),
                ])
            gaths = []
            for b in range(UN):
                for d in loads[b]:
                    d.wait()
                gaths.append([
                    pltpu.async_copy(d0_hbm.at[vr[b]], d0[b], sems[b]),
                    pltpu.async_copy(d1_hbm.at[vr[b]], d1[b], sems[b]),
                    pltpu.async_copy(w1_hbm.at[hi[b]], rows[b], sems[b]),
                ])
            fins = []
            for b in range(UN):
                for d in gaths[b]:
                    d.wait()

                @pl.loop(0, C, step=L)
                def _(i):
                    s = pl.ds(i, L)
                    va[b][s] = 1.0 / (d0[b][s] + d1[b][s])

                fins.append(pltpu.async_copy(
                    va[b], vals_out.at[pl.ds(base0 + b * C, C)], sems[b]))

                @pl.loop(0, C, step=8)
                def _(t):
                    for j in range(8):
                        vv = plsc.load_gather(
                            va[b], [jnp.full((L,), t + j, jnp.int32)])
                        rows[b][t + j, :] = rows[b][t + j, :] * vv

                fins.append(pltpu.async_copy(
                    rows[b], h_sh.at[si[b]], sems[b], add=True))
            for d in fins:
                d.wait()

        plsc.subcore_barrier()
        for off, sz in _pieces(PS, C):
            pltpu.sync_copy(h_sh.at[pl.ds(sid * PS + off, sz), :],
                            rows[0].at[pl.ds(0, sz), :])
            pltpu.sync_copy(rows[0].at[pl.ds(0, sz), :],
                            h_out.at[pl.ds(cid * NP + sid * PS + off, sz), :])

    call = pl.kernel(
        body,
        out_type=(
            jax.ShapeDtypeStruct((NC * NP, EMB), jnp.float32),
            jax.ShapeDtypeStruct((E,), jnp.float32),
        ),
        mesh=_mesh,
        compiler_params=_cp,
        scratch_types=(
            [pltpu.VMEM_SHARED((NP, EMB), jnp.float32)]
            + [pltpu.VMEM((C,), jnp.int32) for _ in range(3 * UN)]
            + [pltpu.VMEM((C,), jnp.float32) for _ in range(3 * UN)]
            + [pltpu.VMEM((C, EMB), jnp.float32) for _ in range(UN)]
            + [pltpu.SemaphoreType.DMA for _ in range(UN)]
        ),
    )
    return call(vrow, hidx, srcw, deg0, deg1, w1, zeros_h)


def _g_call(h_parts, b1, W, N, NP, EMB, RW):
    BN = 2000
    assert N % BN == 0 and RW % 128 == 0
    NQ = RW // 128

    def body(h_ref, b1_ref, w_ref, g_ref):
        h = h_ref[0] + h_ref[1] + b1_ref[...]
        h = jnp.maximum(h, 0.0)
        g_ref[...] = lax.dot_general(
            h, w_ref[...], (((1,), (0,)), ((), ())),
            precision=lax.Precision.HIGHEST,
            preferred_element_type=jnp.float32,
        )

    return pl.pallas_call(
        body,
        grid=(NQ, N // BN),
        in_specs=[
            pl.BlockSpec((NC, BN, EMB), lambda q, i: (0, i, 0)),
            pl.BlockSpec((1, EMB), lambda q, i: (0, 0)),
            pl.BlockSpec((EMB, 128), lambda q, i: (0, q)),
        ],
        out_specs=pl.BlockSpec((BN, 128), lambda q, i: (q * (N // BN) + i, 0)),
        out_shape=jax.ShapeDtypeStruct((NQ * N, 128), jnp.float32),
    )(h_parts, b1, W)


def _out_call(gidx, srcw, vals, g2, zeros_o, NP, E):
    EPW = E // NW
    C, UN, OUT = _chunk_unroll(EPW, 400, 5)
    PS = NP // NS

    def body(gidx_hbm, src_hbm, vals_hbm, g_hbm, zo_hbm,
             o_out,
             o_sh, *rest):
        gi = rest[0:UN]
        si = rest[UN:2 * UN]
        va = rest[2 * UN:3 * UN]
        rows = rest[3 * UN:4 * UN]
        sems = rest[4 * UN:]
        cid, sid, wid = _worker()
        for off, sz in _pieces(PS, C):
            pltpu.sync_copy(zo_hbm.at[pl.ds(sid * PS + off, sz), :],
                            rows[0].at[pl.ds(0, sz), :])
            pltpu.sync_copy(rows[0].at[pl.ds(0, sz), :],
                            o_sh.at[pl.ds(sid * PS + off, sz), :])
        plsc.subcore_barrier()

        @pl.loop(0, OUT)
        def _(g):
            base0 = wid * EPW + g * (UN * C)
            loads = []
            for b in range(UN):
                s = pl.ds(base0 + b * C, C)
                loads.append([
                    pltpu.async_copy(gidx_hbm.at[s], gi[b], sems[b]),
                    pltpu.async_copy(src_hbm.at[s], si[b], sems[b]),
                    pltpu.async_copy(vals_hbm.at[s], va[b], sems[b]),
                ])
            gaths = []
            for b in range(UN):
                for d in loads[b]:
                    d.wait()
                gaths.append(
                    pltpu.async_copy(g_hbm.at[gi[b]], rows[b], sems[b]))
            fins = []
            for b in range(UN):
                gaths[b].wait()

                @pl.loop(0, C, step=8)
                def _(t):
                    for j in range(8):
                        vv = plsc.load_gather(
                            va[b], [jnp.full((L,), t + j, jnp.int32)])
                        rows[b][t + j, :] = rows[b][t + j, :] * vv

                fins.append(pltpu.async_copy(
                    rows[b], o_sh.at[si[b]], sems[b], add=True))
            for d in fins:
                d.wait()

        plsc.subcore_barrier()
        for off, sz in _pieces(PS, C):
            pltpu.sync_copy(o_sh.at[pl.ds(sid * PS + off, sz), :],
                            rows[0].at[pl.ds(0, sz), :])
            pltpu.sync_copy(rows[0].at[pl.ds(0, sz), :],
                            o_out.at[pl.ds(cid * NP + sid * PS + off, sz), :])

    call = pl.kernel(
        body,
        out_type=jax.ShapeDtypeStruct((NC * NP, PADC), jnp.float32),
        mesh=_mesh,
        compiler_params=_cp,
        scratch_types=(
            [pltpu.VMEM_SHARED((NP, PADC), jnp.float32)]
            + [pltpu.VMEM((C,), jnp.int32) for _ in range(2 * UN)]
            + [pltpu.VMEM((C,), jnp.float32) for _ in range(UN)]
            + [pltpu.VMEM((C, PADC), jnp.float32) for _ in range(UN)]
            + [pltpu.SemaphoreType.DMA for _ in range(UN)]
        ),
    )
    return call(gidx, srcw, vals, g2, zeros_o)


def kernel(src, dst, rel, weights1, bias1, weights2, bias2):
    R, N, EMB = weights1.shape
    NUMCLS = weights2.shape[-1]
    E = src.shape[0]
    RN = R * N
    NP = _pad_rows(N)

    src = src.astype(jnp.int32)
    dst = dst.astype(jnp.int32)
    rel = rel.astype(jnp.int32)
    vrow = rel * N + src
    hidx = rel * N + dst
    RPQ = 128 // PADC
    gidx = (rel // RPQ) * (N * RPQ) + dst * RPQ + rel % RPQ

    w1 = weights1.reshape(RN, EMB)
    W = jnp.zeros((R, EMB, PADC), jnp.float32).at[:, :, :NUMCLS].set(weights2)
    W = W.transpose(1, 0, 2).reshape(EMB, R * PADC)

    zeros_rn = jnp.zeros((RN,), jnp.float32)
    zeros_h = jnp.zeros((NP, EMB), jnp.float32)
    zeros_o = jnp.zeros((NP, PADC), jnp.float32)

    deg_parts = _deg_call(vrow, zeros_rn, RN, E)
    h_parts, vals = _h_call(vrow, hidx, src, deg_parts[:RN], deg_parts[RN:],
                            w1, zeros_h, NP, E, EMB)
    hp = h_parts.reshape(NC, NP, EMB)
    g = _g_call(hp, bias1.reshape(1, EMB), W, N, NP, EMB, R * PADC)
    g2 = g.reshape(N * R, PADC)
    o_parts = _out_call(gidx, src, vals, g2, zeros_o, NP, E)
    op = o_parts.reshape(NC, NP, PADC)
    out = op[0, :N, :NUMCLS] + op[1, :N, :NUMCLS] + bias2
    return out
